# baseline (speedup 1.0000x reference)
import functools

import jax
import jax.numpy as jnp
from jax import lax
from jax.experimental import pallas as pl
from jax.experimental.pallas import tpu as pltpu
from jax.experimental.pallas import tpu_sc as plsc

N = 10000
NPAD = 10240
R = 256
CBLK = 1024
K = 6
KPAD = 8
BIGF = 1e9
POISON = 1e18
SC_CORES = 2
SC_SUBCORES = 16
SC_CHUNK = 128


def _insert_sorted(Ms, Is, cand_v, cand_i):
    new_v, new_i = cand_v, cand_i
    last = len(Ms) - 1
    for j in range(len(Ms)):
        mj, ij = Ms[j], Is[j]
        swap = new_v < mj
        Ms[j] = jnp.where(swap, new_v, mj)
        Is[j] = jnp.where(swap, new_i, ij)
        if j != last:
            new_v = jnp.where(swap, mj, new_v)
            new_i = jnp.where(swap, ij, new_i)
    return Ms, Is


def _knn_ab_body(x_ref, xt_ref, w1_ref, b1_ref, idx_ref, a_ref, b_ref,
                 sq_scr, *, d_in, o1, o1g):
    i = pl.program_id(0)
    row0 = pl.multiple_of(i * R, R)
    x_r = x_ref[pl.ds(row0, R), :]
    x_r_bf = x_r.astype(jnp.bfloat16)
    xt_r = xt_ref[:, pl.ds(row0, R)]
    sq_r_row = jnp.sum(xt_r * xt_r, axis=0, keepdims=True)
    row_f = (lax.convert_element_type(i * R, jnp.float32)
             + lax.broadcasted_iota(jnp.int32, (1, R), 1
                                    ).astype(jnp.float32))
    sub8 = lax.broadcasted_iota(jnp.int32, (8, R), 0).astype(jnp.float32)

    @pl.when(i == 0)
    def _fill_sq():
        xf = x_ref[...]
        sq_scr[...] = jnp.sum(xf * xf, axis=1, keepdims=True)

    def make_d_t(c):
        off = pl.multiple_of(c * CBLK, CBLK)
        x_c_bf = x_ref[pl.ds(off, CBLK), :].astype(jnp.bfloat16)
        dots = lax.dot_general(x_c_bf, x_r_bf, (((1,), (1,)), ((), ())),
                               preferred_element_type=jnp.float32)
        sq_c = sq_scr[pl.ds(off, CBLK), :]
        return (sq_c + sq_r_row) - 2.0 * dots

    def sweep(depth):
        def chunk_body(c, carry):
            Ms, Is = carry
            Ms, Is = list(Ms), list(Is)
            d_t = make_d_t(c)
            basef = lax.convert_element_type(c * CBLK, jnp.float32)
            for t in range(CBLK // 8):
                cv = lax.slice(d_t, (8 * t, 0), (8 * t + 8, R))
                ci = sub8 + (basef + float(8 * t))
                cv = jnp.where(ci == row_f, jnp.inf, cv)
                Ms, Is = _insert_sorted(Ms, Is, cv, ci)
            return tuple(Ms), tuple(Is)

        Ms0 = tuple(jnp.full((8, R), jnp.inf, jnp.float32)
                    for _ in range(depth))
        Is0 = tuple(jnp.zeros((8, R), jnp.float32) for _ in range(depth))
        return lax.fori_loop(0, NPAD // CBLK, chunk_body, (Ms0, Is0))

    def merge(Ms, Is):
        V = jnp.concatenate(Ms, axis=0)
        Ic = jnp.concatenate(Is, axis=0)
        out_v, out_i = [], []
        for _ in range(K):
            m = jnp.min(V, axis=0, keepdims=True)
            am = jnp.min(jnp.where(V == m, Ic, jnp.float32(BIGF)),
                         axis=0, keepdims=True)
            am = jnp.minimum(am, jnp.float32(NPAD - 1))
            out_v.append(m)
            out_i.append(am)
            V = jnp.where(Ic == am, jnp.inf, V)
        return out_v, out_i

    def write_idx(out_i):
        outs = out_i + [out_i[-1], out_i[-1]]
        idx_ref[...] = jnp.concatenate(outs, axis=0).astype(jnp.int32)[None]

    Ms, Is = sweep(2)
    out_v, out_i = merge(Ms, Is)
    write_idx(out_i)

    t6 = out_v[K - 1]
    n_sel = jnp.zeros((1, R), jnp.float32)
    for j in range(K):
        n_sel = n_sel + jnp.where(
            out_v[j] < t6, 1.0, jnp.where(out_v[j] == t6, 0.5, 0.0))

    def count_body(c, acc):
        d_t = make_d_t(c)
        basef = lax.convert_element_type(c * CBLK, jnp.float32)
        col0 = lax.broadcasted_iota(jnp.int32, (CBLK, 1), 0
                                    ).astype(jnp.float32) + basef
        d_t = jnp.where(col0 == row_f, jnp.inf, d_t)
        cmb = jnp.where(d_t < t6, 1.0,
                        jnp.where(d_t == t6, 0.5, 0.0))
        return acc + jnp.sum(cmb, axis=0, keepdims=True)

    c_all = lax.fori_loop(0, NPAD // CBLK, count_body,
                          jnp.zeros((1, R), jnp.float32))
    bad = (c_all != n_sel) & (row_f < jnp.float32(N))
    n_bad = jnp.sum(jnp.where(bad, 1.0, 0.0))

    @pl.when(n_bad > 0.0)
    def _exact_fallback():
        Ms6, Is6 = sweep(K)
        _, out_i6 = merge(Ms6, Is6)
        write_idx(out_i6)

    wd = w1_ref[0:d_in, :] - w1_ref[d_in:2 * d_in, :]
    wb = w1_ref[d_in:2 * d_in, :]
    a_ref[...] = (jnp.dot(x_r, wd, preferred_element_type=jnp.float32,
                  precision=lax.Precision.HIGHEST)
                  + b1_ref[...])
    bm = jnp.dot(x_r, wb, preferred_element_type=jnp.float32,
                  precision=lax.Precision.HIGHEST)
    if o1g > o1:
        bm = jnp.concatenate(
            [bm, jnp.zeros((R, o1g - o1), jnp.float32)], axis=1)
    b_ref[...] = bm


def _knn_ab(x_p, x_t, w1, b1, d_in, o1, o1g, interpret=False):
    body = functools.partial(_knn_ab_body, d_in=d_in, o1=o1, o1g=o1g)
    return pl.pallas_call(
        body,
        grid=(NPAD // R,),
        in_specs=[
            pl.BlockSpec((NPAD, d_in), lambda i: (0, 0)),
            pl.BlockSpec((d_in, NPAD), lambda i: (0, 0)),
            pl.BlockSpec((2 * d_in, o1), lambda i: (0, 0)),
            pl.BlockSpec((1, o1), lambda i: (0, 0)),
        ],
        out_specs=[
            pl.BlockSpec((1, KPAD, R), lambda i: (i, 0, 0)),
            pl.BlockSpec((R, o1), lambda i: (i, 0)),
            pl.BlockSpec((R, o1g), lambda i: (i, 0)),
        ],
        out_shape=[
            jax.ShapeDtypeStruct((NPAD // R, KPAD, R), jnp.int32),
            jax.ShapeDtypeStruct((NPAD, o1), jnp.float32),
            jax.ShapeDtypeStruct((NPAD, o1g), jnp.float32),
        ],
        scratch_shapes=[pltpu.VMEM((NPAD, 1), jnp.float32)],
        interpret=interpret,
    )(x_p, x_t, w1, b1.reshape(1, o1))


def _sc_gather(b_mat, idx_flat, o1):
    nidx = idx_flat.shape[0]
    nw = SC_CORES * SC_SUBCORES
    per_w = nidx // nw
    n_chunks = per_w // SC_CHUNK
    mesh = plsc.VectorSubcoreMesh(core_axis_name="c", subcore_axis_name="s")

    @functools.partial(
        pl.kernel, mesh=mesh,
        out_type=jax.ShapeDtypeStruct((nidx, o1), jnp.float32),
        scratch_types=[
            pltpu.VMEM((per_w,), jnp.int32),
            pltpu.VMEM((SC_CHUNK, o1), jnp.float32),
            pltpu.VMEM((SC_CHUNK, o1), jnp.float32),
            pltpu.SemaphoreType.DMA,
            pltpu.SemaphoreType.DMA,
        ],
    )
    def gk(b_hbm, idx_hbm, out_hbm, idx_v, rows_v0, rows_v1, sem_g, sem_w):
        wid = lax.axis_index("s") * SC_CORES + lax.axis_index("c")
        base = wid * per_w
        pltpu.sync_copy(idx_hbm.at[pl.ds(base, per_w)], idx_v)
        bufs = (rows_v0, rows_v1)
        gathers = [None] * n_chunks
        writes = [None] * n_chunks
        gathers[0] = pltpu.async_copy(
            b_hbm.at[idx_v.at[pl.ds(0, SC_CHUNK)]], bufs[0], sem_g)
        for t in range(n_chunks):
            if t + 1 < n_chunks:
                if t >= 1:
                    writes[t - 1].wait()
                gathers[t + 1] = pltpu.async_copy(
                    b_hbm.at[idx_v.at[pl.ds((t + 1) * SC_CHUNK, SC_CHUNK)]],
                    bufs[(t + 1) % 2], sem_g)
            gathers[t].wait()
            writes[t] = pltpu.async_copy(
                bufs[t % 2], out_hbm.at[pl.ds(base + t * SC_CHUNK, SC_CHUNK)],
                sem_w)
        writes[n_chunks - 2].wait()
        writes[n_chunks - 1].wait()

    return gk(b_mat, idx_flat)


def _mlp_body(a_ref, g_ref, w2_ref, b2_ref, *rest, has_skip, o1, poison):
    if has_skip:
        s_ref, o_ref = rest
    else:
        (o_ref,) = rest
    a = a_ref[...]
    w2 = w2_ref[...]
    m = None
    for j in range(K):
        t = jnp.maximum(a + g_ref[j][:, 0:o1], 0.0)
        o = jnp.dot(t, w2, preferred_element_type=jnp.float32,
                  precision=lax.Precision.HIGHEST)
        m = o if m is None else jnp.maximum(m, o)
    m = m + b2_ref[...]
    if has_skip:
        m = m + s_ref[...]
    m = jnp.maximum(m, 0.0)
    if poison:
        rows = pl.program_id(0) * R + lax.broadcasted_iota(
            jnp.int32, (R, 1), 0)
        m = jnp.where(rows >= N, POISON, m)
    o_ref[...] = m


def _mlp(a, g, w2, b2, o1, o1g, o2, skip=None, poison=False,
         interpret=False):
    body = functools.partial(_mlp_body, has_skip=skip is not None, o1=o1,
                             poison=poison)
    in_specs = [
        pl.BlockSpec((R, o1), lambda i: (i, 0)),
        pl.BlockSpec((K, R, o1g), lambda i: (0, i, 0)),
        pl.BlockSpec((o1, o2), lambda i: (0, 0)),
        pl.BlockSpec((1, o2), lambda i: (0, 0)),
    ]
    args = [a, g, w2, b2.reshape(1, o2)]
    if skip is not None:
        in_specs.append(pl.BlockSpec((R, o2), lambda i: (i, 0)))
        args.append(skip)
    return pl.pallas_call(
        body,
        grid=(NPAD // R,),
        in_specs=in_specs,
        out_specs=pl.BlockSpec((R, o2), lambda i: (i, 0)),
        out_shape=jax.ShapeDtypeStruct((NPAD, o2), jnp.float32),
        interpret=interpret,
    )(*args)


def kernel(x, batch, W01, b01, W02, b02, W11, b11, W12, b12, W21, b21,
           W22, b22):
    x_p = jnp.pad(x, ((0, NPAD - N), (0, 0)), constant_values=POISON)

    def layer(xin, w1, b1, w2, b2, d_in, o1, o2, skip=None, poison=False):
        o1g = max(o1, 128)
        idx, a, bmat = _knn_ab(xin, xin.T, w1, b1, d_in, o1, o1g)
        idx_flat = idx[:, :K, :].transpose(1, 0, 2).reshape(-1)
        g = _sc_gather(bmat, idx_flat, o1g).reshape(K, NPAD, o1g)
        return _mlp(a, g, w2, b2, o1, o1g, o2, skip=skip, poison=poison)

    x0 = layer(x_p, W01, b01, W02, b02, 128, 128, 128, poison=True)
    x1 = layer(x0, W11, b11, W12, b12, 128, 64, 64, poison=True)
    out = layer(x1, W21, b21, W22, b22, 64, 128, 128, skip=x0)
    return out[:N]


# speedup vs baseline: 1.7606x; 1.7606x over previous
import functools

import jax
import jax.numpy as jnp
from jax import lax
from jax.experimental import pallas as pl
from jax.experimental.pallas import tpu as pltpu
from jax.experimental.pallas import tpu_sc as plsc

N = 10000
NPAD = 10240
R = 256
CBLK = 1024
K = 6
KPAD = 8
BIGF = 1e9
POISON = 1e18
SC_CORES = 2
SC_SUBCORES = 16
SC_CHUNK = 128


def _insert_sorted(Ms, Is, cand_v, cand_i):
    new_v, new_i = cand_v, cand_i
    last = len(Ms) - 1
    for j in range(len(Ms)):
        mj, ij = Ms[j], Is[j]
        swap = new_v < mj
        Ms[j] = jnp.where(swap, new_v, mj)
        Is[j] = jnp.where(swap, new_i, ij)
        if j != last:
            new_v = jnp.where(swap, mj, new_v)
            new_i = jnp.where(swap, ij, new_i)
    return Ms, Is


def _knn_ab_body(x_ref, xt_ref, w1_ref, b1_ref, idx_ref, a_ref, b_ref,
                 sq_scr, *, d_in, o1, o1g):
    i = pl.program_id(0)
    row0 = pl.multiple_of(i * R, R)
    x_r = x_ref[pl.ds(row0, R), :]
    x_r_bf = x_r.astype(jnp.bfloat16)
    xt_r = xt_ref[:, pl.ds(row0, R)]
    sq_r_row = jnp.sum(xt_r * xt_r, axis=0, keepdims=True)
    row_f = (lax.convert_element_type(i * R, jnp.float32)
             + lax.broadcasted_iota(jnp.int32, (1, R), 1
                                    ).astype(jnp.float32))
    sub8 = lax.broadcasted_iota(jnp.int32, (8, R), 0).astype(jnp.float32)

    @pl.when(i == 0)
    def _fill_sq():
        xf = x_ref[...]
        sq_scr[...] = jnp.sum(xf * xf, axis=1, keepdims=True)

    def make_d_t(c):
        off = pl.multiple_of(c * CBLK, CBLK)
        x_c_bf = x_ref[pl.ds(off, CBLK), :].astype(jnp.bfloat16)
        dots = lax.dot_general(x_c_bf, x_r_bf, (((1,), (1,)), ((), ())),
                               preferred_element_type=jnp.float32)
        sq_c = sq_scr[pl.ds(off, CBLK), :]
        return (sq_c + sq_r_row) - 2.0 * dots

    def sweep(depth):
        def chunk_body(c, carry):
            Ms, Is = carry
            Ms, Is = list(Ms), list(Is)
            d_t = make_d_t(c)
            basef = lax.convert_element_type(c * CBLK, jnp.float32)
            for t in range(CBLK // 8):
                cv = lax.slice(d_t, (8 * t, 0), (8 * t + 8, R))
                ci = sub8 + (basef + float(8 * t))
                cv = jnp.where(ci == row_f, jnp.inf, cv)
                Ms, Is = _insert_sorted(Ms, Is, cv, ci)
            return tuple(Ms), tuple(Is)

        Ms0 = tuple(jnp.full((8, R), jnp.inf, jnp.float32)
                    for _ in range(depth))
        Is0 = tuple(jnp.zeros((8, R), jnp.float32) for _ in range(depth))
        return lax.fori_loop(0, NPAD // CBLK, chunk_body, (Ms0, Is0))

    def merge(Ms, Is):
        V = jnp.concatenate(Ms, axis=0)
        Ic = jnp.concatenate(Is, axis=0)
        out_v, out_i = [], []
        for _ in range(K):
            m = jnp.min(V, axis=0, keepdims=True)
            am = jnp.min(jnp.where(V == m, Ic, jnp.float32(BIGF)),
                         axis=0, keepdims=True)
            am = jnp.minimum(am, jnp.float32(NPAD - 1))
            out_v.append(m)
            out_i.append(am)
            V = jnp.where(Ic == am, jnp.inf, V)
        return out_v, out_i

    def write_idx(out_i):
        outs = out_i + [out_i[-1], out_i[-1]]
        idx_ref[...] = jnp.concatenate(outs, axis=0).astype(jnp.int32)[None]

    Ms, Is = sweep(K)
    _, out_i = merge(Ms, Is)
    write_idx(out_i)

    wd = w1_ref[0:d_in, :] - w1_ref[d_in:2 * d_in, :]
    wb = w1_ref[d_in:2 * d_in, :]
    a_ref[...] = (jnp.dot(x_r, wd, preferred_element_type=jnp.float32,
                  precision=lax.Precision.HIGHEST)
                  + b1_ref[...])
    bm = jnp.dot(x_r, wb, preferred_element_type=jnp.float32,
                  precision=lax.Precision.HIGHEST)
    if o1g > o1:
        bm = jnp.concatenate(
            [bm, jnp.zeros((R, o1g - o1), jnp.float32)], axis=1)
    b_ref[...] = bm


def _knn_ab(x_p, x_t, w1, b1, d_in, o1, o1g, interpret=False):
    body = functools.partial(_knn_ab_body, d_in=d_in, o1=o1, o1g=o1g)
    return pl.pallas_call(
        body,
        grid=(NPAD // R,),
        in_specs=[
            pl.BlockSpec((NPAD, d_in), lambda i: (0, 0)),
            pl.BlockSpec((d_in, NPAD), lambda i: (0, 0)),
            pl.BlockSpec((2 * d_in, o1), lambda i: (0, 0)),
            pl.BlockSpec((1, o1), lambda i: (0, 0)),
        ],
        out_specs=[
            pl.BlockSpec((1, KPAD, R), lambda i: (i, 0, 0)),
            pl.BlockSpec((R, o1), lambda i: (i, 0)),
            pl.BlockSpec((R, o1g), lambda i: (i, 0)),
        ],
        out_shape=[
            jax.ShapeDtypeStruct((NPAD // R, KPAD, R), jnp.int32),
            jax.ShapeDtypeStruct((NPAD, o1), jnp.float32),
            jax.ShapeDtypeStruct((NPAD, o1g), jnp.float32),
        ],
        scratch_shapes=[pltpu.VMEM((NPAD, 1), jnp.float32)],
        interpret=interpret,
    )(x_p, x_t, w1, b1.reshape(1, o1))


def _sc_gather(b_mat, idx_flat, o1):
    nidx = idx_flat.shape[0]
    nw = SC_CORES * SC_SUBCORES
    per_w = nidx // nw
    n_chunks = per_w // SC_CHUNK
    mesh = plsc.VectorSubcoreMesh(core_axis_name="c", subcore_axis_name="s")

    @functools.partial(
        pl.kernel, mesh=mesh,
        out_type=jax.ShapeDtypeStruct((nidx, o1), jnp.float32),
        scratch_types=[
            pltpu.VMEM((per_w,), jnp.int32),
            pltpu.VMEM((SC_CHUNK, o1), jnp.float32),
            pltpu.VMEM((SC_CHUNK, o1), jnp.float32),
            pltpu.SemaphoreType.DMA,
            pltpu.SemaphoreType.DMA,
        ],
    )
    def gk(b_hbm, idx_hbm, out_hbm, idx_v, rows_v0, rows_v1, sem_g, sem_w):
        wid = lax.axis_index("s") * SC_CORES + lax.axis_index("c")
        base = wid * per_w
        pltpu.sync_copy(idx_hbm.at[pl.ds(base, per_w)], idx_v)
        bufs = (rows_v0, rows_v1)
        gathers = [None] * n_chunks
        writes = [None] * n_chunks
        gathers[0] = pltpu.async_copy(
            b_hbm.at[idx_v.at[pl.ds(0, SC_CHUNK)]], bufs[0], sem_g)
        for t in range(n_chunks):
            if t + 1 < n_chunks:
                if t >= 1:
                    writes[t - 1].wait()
                gathers[t + 1] = pltpu.async_copy(
                    b_hbm.at[idx_v.at[pl.ds((t + 1) * SC_CHUNK, SC_CHUNK)]],
                    bufs[(t + 1) % 2], sem_g)
            gathers[t].wait()
            writes[t] = pltpu.async_copy(
                bufs[t % 2], out_hbm.at[pl.ds(base + t * SC_CHUNK, SC_CHUNK)],
                sem_w)
        writes[n_chunks - 2].wait()
        writes[n_chunks - 1].wait()

    return gk(b_mat, idx_flat)


def _mlp_body(a_ref, g_ref, w2_ref, b2_ref, *rest, has_skip, o1, poison):
    if has_skip:
        s_ref, o_ref = rest
    else:
        (o_ref,) = rest
    a = a_ref[...]
    w2 = w2_ref[...]
    m = None
    for j in range(K):
        t = jnp.maximum(a + g_ref[j][:, 0:o1], 0.0)
        o = jnp.dot(t, w2, preferred_element_type=jnp.float32,
                  precision=lax.Precision.HIGHEST)
        m = o if m is None else jnp.maximum(m, o)
    m = m + b2_ref[...]
    if has_skip:
        m = m + s_ref[...]
    m = jnp.maximum(m, 0.0)
    if poison:
        rows = pl.program_id(0) * R + lax.broadcasted_iota(
            jnp.int32, (R, 1), 0)
        m = jnp.where(rows >= N, POISON, m)
    o_ref[...] = m


def _mlp(a, g, w2, b2, o1, o1g, o2, skip=None, poison=False,
         interpret=False):
    body = functools.partial(_mlp_body, has_skip=skip is not None, o1=o1,
                             poison=poison)
    in_specs = [
        pl.BlockSpec((R, o1), lambda i: (i, 0)),
        pl.BlockSpec((K, R, o1g), lambda i: (0, i, 0)),
        pl.BlockSpec((o1, o2), lambda i: (0, 0)),
        pl.BlockSpec((1, o2), lambda i: (0, 0)),
    ]
    args = [a, g, w2, b2.reshape(1, o2)]
    if skip is not None:
        in_specs.append(pl.BlockSpec((R, o2), lambda i: (i, 0)))
        args.append(skip)
    return pl.pallas_call(
        body,
        grid=(NPAD // R,),
        in_specs=in_specs,
        out_specs=pl.BlockSpec((R, o2), lambda i: (i, 0)),
        out_shape=jax.ShapeDtypeStruct((NPAD, o2), jnp.float32),
        interpret=interpret,
    )(*args)


def kernel(x, batch, W01, b01, W02, b02, W11, b11, W12, b12, W21, b21,
           W22, b22):
    x_p = jnp.pad(x, ((0, NPAD - N), (0, 0)), constant_values=POISON)

    def layer(xin, w1, b1, w2, b2, d_in, o1, o2, skip=None, poison=False):
        o1g = max(o1, 128)
        idx, a, bmat = _knn_ab(xin, xin.T, w1, b1, d_in, o1, o1g)
        idx_flat = idx[:, :K, :].transpose(1, 0, 2).reshape(-1)
        g = _sc_gather(bmat, idx_flat, o1g).reshape(K, NPAD, o1g)
        return _mlp(a, g, w2, b2, o1, o1g, o2, skip=skip, poison=poison)

    x0 = layer(x_p, W01, b01, W02, b02, 128, 128, 128, poison=True)
    x1 = layer(x0, W11, b11, W12, b12, 128, 64, 64, poison=True)
    out = layer(x1, W21, b21, W22, b22, 64, 128, 128, skip=x0)
    return out[:N]


# speedup vs baseline: 1.9213x; 1.0913x over previous
import functools

import jax
import jax.numpy as jnp
from jax import lax
from jax.experimental import pallas as pl
from jax.experimental.pallas import tpu as pltpu
from jax.experimental.pallas import tpu_sc as plsc

N = 10000
NPAD = 10240
R = 512
CBLK = 1024
K = 6
KPAD = 8
BIGF = 1e9
POISON = 1e18
SC_CORES = 2
SC_SUBCORES = 16
SC_CHUNK = 128


def _insert_sorted(Ms, Is, cand_v, cand_i):
    new_v, new_i = cand_v, cand_i
    last = len(Ms) - 1
    for j in range(len(Ms)):
        mj, ij = Ms[j], Is[j]
        swap = new_v < mj
        Ms[j] = jnp.where(swap, new_v, mj)
        Is[j] = jnp.where(swap, new_i, ij)
        if j != last:
            new_v = jnp.where(swap, mj, new_v)
            new_i = jnp.where(swap, ij, new_i)
    return Ms, Is


def _knn_ab_body(x_ref, xt_ref, w1_ref, b1_ref, idx_ref, a_ref, b_ref,
                 sq_scr, *, d_in, o1, o1g):
    i = pl.program_id(0)
    row0 = pl.multiple_of(i * R, R)
    x_r = x_ref[pl.ds(row0, R), :]
    x_r_bf = x_r.astype(jnp.bfloat16)
    xt_r = xt_ref[:, pl.ds(row0, R)]
    sq_r_row = jnp.sum(xt_r * xt_r, axis=0, keepdims=True)
    row_f = (lax.convert_element_type(i * R, jnp.float32)
             + lax.broadcasted_iota(jnp.int32, (1, R), 1
                                    ).astype(jnp.float32))
    sub8 = lax.broadcasted_iota(jnp.int32, (8, R), 0).astype(jnp.float32)

    @pl.when(i == 0)
    def _fill_sq():
        xf = x_ref[...]
        sq_scr[...] = jnp.sum(xf * xf, axis=1, keepdims=True)

    def make_d_t(c):
        off = pl.multiple_of(c * CBLK, CBLK)
        x_c_bf = x_ref[pl.ds(off, CBLK), :].astype(jnp.bfloat16)
        dots = lax.dot_general(x_c_bf, x_r_bf, (((1,), (1,)), ((), ())),
                               preferred_element_type=jnp.float32)
        sq_c = sq_scr[pl.ds(off, CBLK), :]
        return (sq_c + sq_r_row) - 2.0 * dots

    def sweep(depth):
        def chunk_body(c, carry):
            Ms, Is = carry
            Ms, Is = list(Ms), list(Is)
            d_t = make_d_t(c)
            basef = lax.convert_element_type(c * CBLK, jnp.float32)
            for t in range(CBLK // 8):
                cv = lax.slice(d_t, (8 * t, 0), (8 * t + 8, R))
                ci = sub8 + (basef + float(8 * t))
                cv = jnp.where(ci == row_f, jnp.inf, cv)
                Ms, Is = _insert_sorted(Ms, Is, cv, ci)
            return tuple(Ms), tuple(Is)

        Ms0 = tuple(jnp.full((8, R), jnp.inf, jnp.float32)
                    for _ in range(depth))
        Is0 = tuple(jnp.zeros((8, R), jnp.float32) for _ in range(depth))
        return lax.fori_loop(0, NPAD // CBLK, chunk_body, (Ms0, Is0))

    def merge(Ms, Is):
        V = jnp.concatenate(Ms, axis=0)
        Ic = jnp.concatenate(Is, axis=0)
        out_v, out_i = [], []
        for _ in range(K):
            m = jnp.min(V, axis=0, keepdims=True)
            am = jnp.min(jnp.where(V == m, Ic, jnp.float32(BIGF)),
                         axis=0, keepdims=True)
            am = jnp.minimum(am, jnp.float32(NPAD - 1))
            out_v.append(m)
            out_i.append(am)
            V = jnp.where(Ic == am, jnp.inf, V)
        return out_v, out_i

    def write_idx(out_i):
        outs = out_i + [out_i[-1], out_i[-1]]
        idx_ref[...] = jnp.concatenate(outs, axis=0).astype(jnp.int32)[None]

    Ms, Is = sweep(K)
    _, out_i = merge(Ms, Is)
    write_idx(out_i)

    wd = w1_ref[0:d_in, :] - w1_ref[d_in:2 * d_in, :]
    wb = w1_ref[d_in:2 * d_in, :]
    a_ref[...] = (jnp.dot(x_r, wd, preferred_element_type=jnp.float32,
                  precision=lax.Precision.HIGHEST)
                  + b1_ref[...])
    bm = jnp.dot(x_r, wb, preferred_element_type=jnp.float32,
                  precision=lax.Precision.HIGHEST)
    if o1g > o1:
        bm = jnp.concatenate(
            [bm, jnp.zeros((R, o1g - o1), jnp.float32)], axis=1)
    b_ref[...] = bm


def _knn_ab(x_p, x_t, w1, b1, d_in, o1, o1g, interpret=False):
    body = functools.partial(_knn_ab_body, d_in=d_in, o1=o1, o1g=o1g)
    return pl.pallas_call(
        body,
        grid=(NPAD // R,),
        in_specs=[
            pl.BlockSpec((NPAD, d_in), lambda i: (0, 0)),
            pl.BlockSpec((d_in, NPAD), lambda i: (0, 0)),
            pl.BlockSpec((2 * d_in, o1), lambda i: (0, 0)),
            pl.BlockSpec((1, o1), lambda i: (0, 0)),
        ],
        out_specs=[
            pl.BlockSpec((1, KPAD, R), lambda i: (i, 0, 0)),
            pl.BlockSpec((R, o1), lambda i: (i, 0)),
            pl.BlockSpec((R, o1g), lambda i: (i, 0)),
        ],
        out_shape=[
            jax.ShapeDtypeStruct((NPAD // R, KPAD, R), jnp.int32),
            jax.ShapeDtypeStruct((NPAD, o1), jnp.float32),
            jax.ShapeDtypeStruct((NPAD, o1g), jnp.float32),
        ],
        scratch_shapes=[pltpu.VMEM((NPAD, 1), jnp.float32)],
        interpret=interpret,
    )(x_p, x_t, w1, b1.reshape(1, o1))


def _sc_gather(b_mat, idx_flat, o1):
    nidx = idx_flat.shape[0]
    nw = SC_CORES * SC_SUBCORES
    per_w = nidx // nw
    n_chunks = per_w // SC_CHUNK
    mesh = plsc.VectorSubcoreMesh(core_axis_name="c", subcore_axis_name="s")

    @functools.partial(
        pl.kernel, mesh=mesh,
        out_type=jax.ShapeDtypeStruct((nidx, o1), jnp.float32),
        scratch_types=[
            pltpu.VMEM((per_w,), jnp.int32),
            pltpu.VMEM((SC_CHUNK, o1), jnp.float32),
            pltpu.VMEM((SC_CHUNK, o1), jnp.float32),
            pltpu.SemaphoreType.DMA,
            pltpu.SemaphoreType.DMA,
        ],
    )
    def gk(b_hbm, idx_hbm, out_hbm, idx_v, rows_v0, rows_v1, sem_g, sem_w):
        wid = lax.axis_index("s") * SC_CORES + lax.axis_index("c")
        base = wid * per_w
        pltpu.sync_copy(idx_hbm.at[pl.ds(base, per_w)], idx_v)
        bufs = (rows_v0, rows_v1)
        gathers = [None] * n_chunks
        writes = [None] * n_chunks
        gathers[0] = pltpu.async_copy(
            b_hbm.at[idx_v.at[pl.ds(0, SC_CHUNK)]], bufs[0], sem_g)
        for t in range(n_chunks):
            if t + 1 < n_chunks:
                if t >= 1:
                    writes[t - 1].wait()
                gathers[t + 1] = pltpu.async_copy(
                    b_hbm.at[idx_v.at[pl.ds((t + 1) * SC_CHUNK, SC_CHUNK)]],
                    bufs[(t + 1) % 2], sem_g)
            gathers[t].wait()
            writes[t] = pltpu.async_copy(
                bufs[t % 2], out_hbm.at[pl.ds(base + t * SC_CHUNK, SC_CHUNK)],
                sem_w)
        writes[n_chunks - 2].wait()
        writes[n_chunks - 1].wait()

    return gk(b_mat, idx_flat)


def _mlp_body(a_ref, g_ref, w2_ref, b2_ref, *rest, has_skip, o1, poison):
    if has_skip:
        s_ref, o_ref = rest
    else:
        (o_ref,) = rest
    a = a_ref[...]
    w2 = w2_ref[...]
    m = None
    for j in range(K):
        t = jnp.maximum(a + g_ref[j][:, 0:o1], 0.0)
        o = jnp.dot(t, w2, preferred_element_type=jnp.float32,
                  precision=lax.Precision.HIGHEST)
        m = o if m is None else jnp.maximum(m, o)
    m = m + b2_ref[...]
    if has_skip:
        m = m + s_ref[...]
    m = jnp.maximum(m, 0.0)
    if poison:
        rows = pl.program_id(0) * R + lax.broadcasted_iota(
            jnp.int32, (R, 1), 0)
        m = jnp.where(rows >= N, POISON, m)
    o_ref[...] = m


def _mlp(a, g, w2, b2, o1, o1g, o2, skip=None, poison=False,
         interpret=False):
    body = functools.partial(_mlp_body, has_skip=skip is not None, o1=o1,
                             poison=poison)
    in_specs = [
        pl.BlockSpec((R, o1), lambda i: (i, 0)),
        pl.BlockSpec((K, R, o1g), lambda i: (0, i, 0)),
        pl.BlockSpec((o1, o2), lambda i: (0, 0)),
        pl.BlockSpec((1, o2), lambda i: (0, 0)),
    ]
    args = [a, g, w2, b2.reshape(1, o2)]
    if skip is not None:
        in_specs.append(pl.BlockSpec((R, o2), lambda i: (i, 0)))
        args.append(skip)
    return pl.pallas_call(
        body,
        grid=(NPAD // R,),
        in_specs=in_specs,
        out_specs=pl.BlockSpec((R, o2), lambda i: (i, 0)),
        out_shape=jax.ShapeDtypeStruct((NPAD, o2), jnp.float32),
        interpret=interpret,
    )(*args)


def kernel(x, batch, W01, b01, W02, b02, W11, b11, W12, b12, W21, b21,
           W22, b22):
    x_p = jnp.pad(x, ((0, NPAD - N), (0, 0)), constant_values=POISON)

    def layer(xin, w1, b1, w2, b2, d_in, o1, o2, skip=None, poison=False):
        o1g = max(o1, 128)
        idx, a, bmat = _knn_ab(xin, xin.T, w1, b1, d_in, o1, o1g)
        idx_flat = idx[:, :K, :].transpose(1, 0, 2).reshape(-1)
        g = _sc_gather(bmat, idx_flat, o1g).reshape(K, NPAD, o1g)
        return _mlp(a, g, w2, b2, o1, o1g, o2, skip=skip, poison=poison)

    x0 = layer(x_p, W01, b01, W02, b02, 128, 128, 128, poison=True)
    x1 = layer(x0, W11, b11, W12, b12, 128, 64, 64, poison=True)
    out = layer(x1, W21, b21, W22, b22, 64, 128, 128, skip=x0)
    return out[:N]


# speedup vs baseline: 1.9467x; 1.0132x over previous
import functools

import jax
import jax.numpy as jnp
from jax import lax
from jax.experimental import pallas as pl
from jax.experimental.pallas import tpu as pltpu
from jax.experimental.pallas import tpu_sc as plsc

N = 10000
NPAD = 10240
R = 1024
CBLK = 1024
K = 6
KPAD = 8
BIGF = 1e9
POISON = 1e18
SC_CORES = 2
SC_SUBCORES = 16
SC_CHUNK = 128


def _insert_sorted(Ms, Is, cand_v, cand_i):
    new_v, new_i = cand_v, cand_i
    last = len(Ms) - 1
    for j in range(len(Ms)):
        mj, ij = Ms[j], Is[j]
        swap = new_v < mj
        Ms[j] = jnp.where(swap, new_v, mj)
        Is[j] = jnp.where(swap, new_i, ij)
        if j != last:
            new_v = jnp.where(swap, mj, new_v)
            new_i = jnp.where(swap, ij, new_i)
    return Ms, Is


def _knn_ab_body(x_ref, xt_ref, w1_ref, b1_ref, idx_ref, a_ref, b_ref,
                 sq_scr, *, d_in, o1, o1g):
    i = pl.program_id(0)
    row0 = pl.multiple_of(i * R, R)
    x_r = x_ref[pl.ds(row0, R), :]
    x_r_bf = x_r.astype(jnp.bfloat16)
    xt_r = xt_ref[:, pl.ds(row0, R)]
    sq_r_row = jnp.sum(xt_r * xt_r, axis=0, keepdims=True)
    row_f = (lax.convert_element_type(i * R, jnp.float32)
             + lax.broadcasted_iota(jnp.int32, (1, R), 1
                                    ).astype(jnp.float32))
    sub8 = lax.broadcasted_iota(jnp.int32, (8, R), 0).astype(jnp.float32)

    @pl.when(i == 0)
    def _fill_sq():
        xf = x_ref[...]
        sq_scr[...] = jnp.sum(xf * xf, axis=1, keepdims=True)

    def make_d_t(c):
        off = pl.multiple_of(c * CBLK, CBLK)
        x_c_bf = x_ref[pl.ds(off, CBLK), :].astype(jnp.bfloat16)
        dots = lax.dot_general(x_c_bf, x_r_bf, (((1,), (1,)), ((), ())),
                               preferred_element_type=jnp.float32)
        sq_c = sq_scr[pl.ds(off, CBLK), :]
        return (sq_c + sq_r_row) - 2.0 * dots

    def sweep(depth):
        def chunk_body(c, carry):
            Ms, Is = carry
            Ms, Is = list(Ms), list(Is)
            d_t = make_d_t(c)
            basef = lax.convert_element_type(c * CBLK, jnp.float32)
            for t in range(CBLK // 8):
                cv = lax.slice(d_t, (8 * t, 0), (8 * t + 8, R))
                ci = sub8 + (basef + float(8 * t))
                cv = jnp.where(ci == row_f, jnp.inf, cv)
                Ms, Is = _insert_sorted(Ms, Is, cv, ci)
            return tuple(Ms), tuple(Is)

        Ms0 = tuple(jnp.full((8, R), jnp.inf, jnp.float32)
                    for _ in range(depth))
        Is0 = tuple(jnp.zeros((8, R), jnp.float32) for _ in range(depth))
        return lax.fori_loop(0, NPAD // CBLK, chunk_body, (Ms0, Is0))

    def merge(Ms, Is):
        V = jnp.concatenate(Ms, axis=0)
        Ic = jnp.concatenate(Is, axis=0)
        out_v, out_i = [], []
        for _ in range(K):
            m = jnp.min(V, axis=0, keepdims=True)
            am = jnp.min(jnp.where(V == m, Ic, jnp.float32(BIGF)),
                         axis=0, keepdims=True)
            am = jnp.minimum(am, jnp.float32(NPAD - 1))
            out_v.append(m)
            out_i.append(am)
            V = jnp.where(Ic == am, jnp.inf, V)
        return out_v, out_i

    def write_idx(out_i):
        outs = out_i + [out_i[-1], out_i[-1]]
        idx_ref[...] = jnp.concatenate(outs, axis=0).astype(jnp.int32)[None]

    Ms, Is = sweep(K)
    _, out_i = merge(Ms, Is)
    write_idx(out_i)

    wd = w1_ref[0:d_in, :] - w1_ref[d_in:2 * d_in, :]
    wb = w1_ref[d_in:2 * d_in, :]
    a_ref[...] = (jnp.dot(x_r, wd, preferred_element_type=jnp.float32,
                  precision=lax.Precision.HIGHEST)
                  + b1_ref[...])
    bm = jnp.dot(x_r, wb, preferred_element_type=jnp.float32,
                  precision=lax.Precision.HIGHEST)
    if o1g > o1:
        bm = jnp.concatenate(
            [bm, jnp.zeros((R, o1g - o1), jnp.float32)], axis=1)
    b_ref[...] = bm


def _knn_ab(x_p, x_t, w1, b1, d_in, o1, o1g, interpret=False):
    body = functools.partial(_knn_ab_body, d_in=d_in, o1=o1, o1g=o1g)
    return pl.pallas_call(
        body,
        grid=(NPAD // R,),
        in_specs=[
            pl.BlockSpec((NPAD, d_in), lambda i: (0, 0)),
            pl.BlockSpec((d_in, NPAD), lambda i: (0, 0)),
            pl.BlockSpec((2 * d_in, o1), lambda i: (0, 0)),
            pl.BlockSpec((1, o1), lambda i: (0, 0)),
        ],
        out_specs=[
            pl.BlockSpec((1, KPAD, R), lambda i: (i, 0, 0)),
            pl.BlockSpec((R, o1), lambda i: (i, 0)),
            pl.BlockSpec((R, o1g), lambda i: (i, 0)),
        ],
        out_shape=[
            jax.ShapeDtypeStruct((NPAD // R, KPAD, R), jnp.int32),
            jax.ShapeDtypeStruct((NPAD, o1), jnp.float32),
            jax.ShapeDtypeStruct((NPAD, o1g), jnp.float32),
        ],
        scratch_shapes=[pltpu.VMEM((NPAD, 1), jnp.float32)],
        interpret=interpret,
    )(x_p, x_t, w1, b1.reshape(1, o1))


def _sc_gather(b_mat, idx_flat, o1):
    nidx = idx_flat.shape[0]
    nw = SC_CORES * SC_SUBCORES
    per_w = nidx // nw
    n_chunks = per_w // SC_CHUNK
    mesh = plsc.VectorSubcoreMesh(core_axis_name="c", subcore_axis_name="s")

    @functools.partial(
        pl.kernel, mesh=mesh,
        out_type=jax.ShapeDtypeStruct((nidx, o1), jnp.float32),
        scratch_types=[
            pltpu.VMEM((per_w,), jnp.int32),
            pltpu.VMEM((SC_CHUNK, o1), jnp.float32),
            pltpu.VMEM((SC_CHUNK, o1), jnp.float32),
            pltpu.SemaphoreType.DMA,
            pltpu.SemaphoreType.DMA,
        ],
    )
    def gk(b_hbm, idx_hbm, out_hbm, idx_v, rows_v0, rows_v1, sem_g, sem_w):
        wid = lax.axis_index("s") * SC_CORES + lax.axis_index("c")
        base = wid * per_w
        pltpu.sync_copy(idx_hbm.at[pl.ds(base, per_w)], idx_v)
        bufs = (rows_v0, rows_v1)
        gathers = [None] * n_chunks
        writes = [None] * n_chunks
        gathers[0] = pltpu.async_copy(
            b_hbm.at[idx_v.at[pl.ds(0, SC_CHUNK)]], bufs[0], sem_g)
        for t in range(n_chunks):
            if t + 1 < n_chunks:
                if t >= 1:
                    writes[t - 1].wait()
                gathers[t + 1] = pltpu.async_copy(
                    b_hbm.at[idx_v.at[pl.ds((t + 1) * SC_CHUNK, SC_CHUNK)]],
                    bufs[(t + 1) % 2], sem_g)
            gathers[t].wait()
            writes[t] = pltpu.async_copy(
                bufs[t % 2], out_hbm.at[pl.ds(base + t * SC_CHUNK, SC_CHUNK)],
                sem_w)
        writes[n_chunks - 2].wait()
        writes[n_chunks - 1].wait()

    return gk(b_mat, idx_flat)


def _mlp_body(a_ref, g_ref, w2_ref, b2_ref, *rest, has_skip, o1, poison):
    if has_skip:
        s_ref, o_ref = rest
    else:
        (o_ref,) = rest
    a = a_ref[...]
    w2 = w2_ref[...]
    m = None
    for j in range(K):
        t = jnp.maximum(a + g_ref[j][:, 0:o1], 0.0)
        o = jnp.dot(t, w2, preferred_element_type=jnp.float32,
                  precision=lax.Precision.HIGHEST)
        m = o if m is None else jnp.maximum(m, o)
    m = m + b2_ref[...]
    if has_skip:
        m = m + s_ref[...]
    m = jnp.maximum(m, 0.0)
    if poison:
        rows = pl.program_id(0) * R + lax.broadcasted_iota(
            jnp.int32, (R, 1), 0)
        m = jnp.where(rows >= N, POISON, m)
    o_ref[...] = m


def _mlp(a, g, w2, b2, o1, o1g, o2, skip=None, poison=False,
         interpret=False):
    body = functools.partial(_mlp_body, has_skip=skip is not None, o1=o1,
                             poison=poison)
    in_specs = [
        pl.BlockSpec((R, o1), lambda i: (i, 0)),
        pl.BlockSpec((K, R, o1g), lambda i: (0, i, 0)),
        pl.BlockSpec((o1, o2), lambda i: (0, 0)),
        pl.BlockSpec((1, o2), lambda i: (0, 0)),
    ]
    args = [a, g, w2, b2.reshape(1, o2)]
    if skip is not None:
        in_specs.append(pl.BlockSpec((R, o2), lambda i: (i, 0)))
        args.append(skip)
    return pl.pallas_call(
        body,
        grid=(NPAD // R,),
        in_specs=in_specs,
        out_specs=pl.BlockSpec((R, o2), lambda i: (i, 0)),
        out_shape=jax.ShapeDtypeStruct((NPAD, o2), jnp.float32),
        interpret=interpret,
    )(*args)


def kernel(x, batch, W01, b01, W02, b02, W11, b11, W12, b12, W21, b21,
           W22, b22):
    x_p = jnp.pad(x, ((0, NPAD - N), (0, 0)), constant_values=POISON)

    def layer(xin, w1, b1, w2, b2, d_in, o1, o2, skip=None, poison=False):
        o1g = max(o1, 128)
        idx, a, bmat = _knn_ab(xin, xin.T, w1, b1, d_in, o1, o1g)
        idx_flat = idx[:, :K, :].transpose(1, 0, 2).reshape(-1)
        g = _sc_gather(bmat, idx_flat, o1g).reshape(K, NPAD, o1g)
        return _mlp(a, g, w2, b2, o1, o1g, o2, skip=skip, poison=poison)

    x0 = layer(x_p, W01, b01, W02, b02, 128, 128, 128, poison=True)
    x1 = layer(x0, W11, b11, W12, b12, 128, 64, 64, poison=True)
    out = layer(x1, W21, b21, W22, b22, 64, 128, 128, skip=x0)
    return out[:N]


# speedup vs baseline: 1.9957x; 1.0251x over previous
import functools

import jax
import jax.numpy as jnp
from jax import lax
from jax.experimental import pallas as pl
from jax.experimental.pallas import tpu as pltpu
from jax.experimental.pallas import tpu_sc as plsc

N = 10000
NPAD = 10240
R = 1024
CBLK = 2048
K = 6
KPAD = 8
BIGF = 1e9
POISON = 1e18
SC_CORES = 2
SC_SUBCORES = 16
SC_CHUNK = 128


def _insert_sorted(Ms, Is, cand_v, cand_i):
    new_v, new_i = cand_v, cand_i
    last = len(Ms) - 1
    for j in range(len(Ms)):
        mj, ij = Ms[j], Is[j]
        swap = new_v < mj
        Ms[j] = jnp.where(swap, new_v, mj)
        Is[j] = jnp.where(swap, new_i, ij)
        if j != last:
            new_v = jnp.where(swap, mj, new_v)
            new_i = jnp.where(swap, ij, new_i)
    return Ms, Is


def _knn_ab_body(x_ref, xt_ref, w1_ref, b1_ref, idx_ref, a_ref, b_ref,
                 sq_scr, *, d_in, o1, o1g):
    i = pl.program_id(0)
    row0 = pl.multiple_of(i * R, R)
    x_r = x_ref[pl.ds(row0, R), :]
    x_r_bf = x_r.astype(jnp.bfloat16)
    xt_r = xt_ref[:, pl.ds(row0, R)]
    sq_r_row = jnp.sum(xt_r * xt_r, axis=0, keepdims=True)
    row_f = (lax.convert_element_type(i * R, jnp.float32)
             + lax.broadcasted_iota(jnp.int32, (1, R), 1
                                    ).astype(jnp.float32))
    sub8 = lax.broadcasted_iota(jnp.int32, (8, R), 0).astype(jnp.float32)

    @pl.when(i == 0)
    def _fill_sq():
        xf = x_ref[...]
        sq_scr[...] = jnp.sum(xf * xf, axis=1, keepdims=True)

    def make_d_t(c):
        off = pl.multiple_of(c * CBLK, CBLK)
        x_c_bf = x_ref[pl.ds(off, CBLK), :].astype(jnp.bfloat16)
        dots = lax.dot_general(x_c_bf, x_r_bf, (((1,), (1,)), ((), ())),
                               preferred_element_type=jnp.float32)
        sq_c = sq_scr[pl.ds(off, CBLK), :]
        return (sq_c + sq_r_row) - 2.0 * dots

    def sweep(depth):
        def chunk_body(c, carry):
            Ms, Is = carry
            Ms, Is = list(Ms), list(Is)
            d_t = make_d_t(c)
            basef = lax.convert_element_type(c * CBLK, jnp.float32)
            for t in range(CBLK // 8):
                cv = lax.slice(d_t, (8 * t, 0), (8 * t + 8, R))
                ci = sub8 + (basef + float(8 * t))
                cv = jnp.where(ci == row_f, jnp.inf, cv)
                Ms, Is = _insert_sorted(Ms, Is, cv, ci)
            return tuple(Ms), tuple(Is)

        Ms0 = tuple(jnp.full((8, R), jnp.inf, jnp.float32)
                    for _ in range(depth))
        Is0 = tuple(jnp.zeros((8, R), jnp.float32) for _ in range(depth))
        return lax.fori_loop(0, NPAD // CBLK, chunk_body, (Ms0, Is0))

    def merge(Ms, Is):
        V = jnp.concatenate(Ms, axis=0)
        Ic = jnp.concatenate(Is, axis=0)
        out_v, out_i = [], []
        for _ in range(K):
            m = jnp.min(V, axis=0, keepdims=True)
            am = jnp.min(jnp.where(V == m, Ic, jnp.float32(BIGF)),
                         axis=0, keepdims=True)
            am = jnp.minimum(am, jnp.float32(NPAD - 1))
            out_v.append(m)
            out_i.append(am)
            V = jnp.where(Ic == am, jnp.inf, V)
        return out_v, out_i

    def write_idx(out_i):
        outs = out_i + [out_i[-1], out_i[-1]]
        idx_ref[...] = jnp.concatenate(outs, axis=0).astype(jnp.int32)[None]

    Ms, Is = sweep(K)
    _, out_i = merge(Ms, Is)
    write_idx(out_i)

    wd = w1_ref[0:d_in, :] - w1_ref[d_in:2 * d_in, :]
    wb = w1_ref[d_in:2 * d_in, :]
    a_ref[...] = (jnp.dot(x_r, wd, preferred_element_type=jnp.float32,
                  precision=lax.Precision.HIGHEST)
                  + b1_ref[...])
    bm = jnp.dot(x_r, wb, preferred_element_type=jnp.float32,
                  precision=lax.Precision.HIGHEST)
    if o1g > o1:
        bm = jnp.concatenate(
            [bm, jnp.zeros((R, o1g - o1), jnp.float32)], axis=1)
    b_ref[...] = bm


def _knn_ab(x_p, x_t, w1, b1, d_in, o1, o1g, interpret=False):
    body = functools.partial(_knn_ab_body, d_in=d_in, o1=o1, o1g=o1g)
    return pl.pallas_call(
        body,
        grid=(NPAD // R,),
        in_specs=[
            pl.BlockSpec((NPAD, d_in), lambda i: (0, 0)),
            pl.BlockSpec((d_in, NPAD), lambda i: (0, 0)),
            pl.BlockSpec((2 * d_in, o1), lambda i: (0, 0)),
            pl.BlockSpec((1, o1), lambda i: (0, 0)),
        ],
        out_specs=[
            pl.BlockSpec((1, KPAD, R), lambda i: (i, 0, 0)),
            pl.BlockSpec((R, o1), lambda i: (i, 0)),
            pl.BlockSpec((R, o1g), lambda i: (i, 0)),
        ],
        out_shape=[
            jax.ShapeDtypeStruct((NPAD // R, KPAD, R), jnp.int32),
            jax.ShapeDtypeStruct((NPAD, o1), jnp.float32),
            jax.ShapeDtypeStruct((NPAD, o1g), jnp.float32),
        ],
        scratch_shapes=[pltpu.VMEM((NPAD, 1), jnp.float32)],
        interpret=interpret,
    )(x_p, x_t, w1, b1.reshape(1, o1))


def _sc_gather(b_mat, idx_flat, o1):
    nidx = idx_flat.shape[0]
    nw = SC_CORES * SC_SUBCORES
    per_w = nidx // nw
    n_chunks = per_w // SC_CHUNK
    mesh = plsc.VectorSubcoreMesh(core_axis_name="c", subcore_axis_name="s")

    @functools.partial(
        pl.kernel, mesh=mesh,
        out_type=jax.ShapeDtypeStruct((nidx, o1), jnp.float32),
        scratch_types=[
            pltpu.VMEM((per_w,), jnp.int32),
            pltpu.VMEM((SC_CHUNK, o1), jnp.float32),
            pltpu.VMEM((SC_CHUNK, o1), jnp.float32),
            pltpu.SemaphoreType.DMA,
            pltpu.SemaphoreType.DMA,
        ],
    )
    def gk(b_hbm, idx_hbm, out_hbm, idx_v, rows_v0, rows_v1, sem_g, sem_w):
        wid = lax.axis_index("s") * SC_CORES + lax.axis_index("c")
        base = wid * per_w
        pltpu.sync_copy(idx_hbm.at[pl.ds(base, per_w)], idx_v)
        bufs = (rows_v0, rows_v1)
        gathers = [None] * n_chunks
        writes = [None] * n_chunks
        gathers[0] = pltpu.async_copy(
            b_hbm.at[idx_v.at[pl.ds(0, SC_CHUNK)]], bufs[0], sem_g)
        for t in range(n_chunks):
            if t + 1 < n_chunks:
                if t >= 1:
                    writes[t - 1].wait()
                gathers[t + 1] = pltpu.async_copy(
                    b_hbm.at[idx_v.at[pl.ds((t + 1) * SC_CHUNK, SC_CHUNK)]],
                    bufs[(t + 1) % 2], sem_g)
            gathers[t].wait()
            writes[t] = pltpu.async_copy(
                bufs[t % 2], out_hbm.at[pl.ds(base + t * SC_CHUNK, SC_CHUNK)],
                sem_w)
        writes[n_chunks - 2].wait()
        writes[n_chunks - 1].wait()

    return gk(b_mat, idx_flat)


def _mlp_body(a_ref, g_ref, w2_ref, b2_ref, *rest, has_skip, o1, poison):
    if has_skip:
        s_ref, o_ref = rest
    else:
        (o_ref,) = rest
    a = a_ref[...]
    w2 = w2_ref[...]
    m = None
    for j in range(K):
        t = jnp.maximum(a + g_ref[j][:, 0:o1], 0.0)
        o = jnp.dot(t, w2, preferred_element_type=jnp.float32,
                  precision=lax.Precision.HIGHEST)
        m = o if m is None else jnp.maximum(m, o)
    m = m + b2_ref[...]
    if has_skip:
        m = m + s_ref[...]
    m = jnp.maximum(m, 0.0)
    if poison:
        rows = pl.program_id(0) * R + lax.broadcasted_iota(
            jnp.int32, (R, 1), 0)
        m = jnp.where(rows >= N, POISON, m)
    o_ref[...] = m


def _mlp(a, g, w2, b2, o1, o1g, o2, skip=None, poison=False,
         interpret=False):
    body = functools.partial(_mlp_body, has_skip=skip is not None, o1=o1,
                             poison=poison)
    in_specs = [
        pl.BlockSpec((R, o1), lambda i: (i, 0)),
        pl.BlockSpec((K, R, o1g), lambda i: (0, i, 0)),
        pl.BlockSpec((o1, o2), lambda i: (0, 0)),
        pl.BlockSpec((1, o2), lambda i: (0, 0)),
    ]
    args = [a, g, w2, b2.reshape(1, o2)]
    if skip is not None:
        in_specs.append(pl.BlockSpec((R, o2), lambda i: (i, 0)))
        args.append(skip)
    return pl.pallas_call(
        body,
        grid=(NPAD // R,),
        in_specs=in_specs,
        out_specs=pl.BlockSpec((R, o2), lambda i: (i, 0)),
        out_shape=jax.ShapeDtypeStruct((NPAD, o2), jnp.float32),
        interpret=interpret,
    )(*args)


def kernel(x, batch, W01, b01, W02, b02, W11, b11, W12, b12, W21, b21,
           W22, b22):
    x_p = jnp.pad(x, ((0, NPAD - N), (0, 0)), constant_values=POISON)

    def layer(xin, w1, b1, w2, b2, d_in, o1, o2, skip=None, poison=False):
        o1g = max(o1, 128)
        idx, a, bmat = _knn_ab(xin, xin.T, w1, b1, d_in, o1, o1g)
        idx_flat = idx[:, :K, :].transpose(1, 0, 2).reshape(-1)
        g = _sc_gather(bmat, idx_flat, o1g).reshape(K, NPAD, o1g)
        return _mlp(a, g, w2, b2, o1, o1g, o2, skip=skip, poison=poison)

    x0 = layer(x_p, W01, b01, W02, b02, 128, 128, 128, poison=True)
    x1 = layer(x0, W11, b11, W12, b12, 128, 64, 64, poison=True)
    out = layer(x1, W21, b21, W22, b22, 64, 128, 128, skip=x0)
    return out[:N]


# speedup vs baseline: 2.0142x; 1.0093x over previous
import functools

import jax
import jax.numpy as jnp
from jax import lax
from jax.experimental import pallas as pl
from jax.experimental.pallas import tpu as pltpu
from jax.experimental.pallas import tpu_sc as plsc

N = 10000
NPAD = 10240
R = 1024
CBLK = 5120
K = 6
KPAD = 8
BIGF = 1e9
POISON = 1e18
SC_CORES = 2
SC_SUBCORES = 16
SC_CHUNK = 128


def _insert_sorted(Ms, Is, cand_v, cand_i):
    new_v, new_i = cand_v, cand_i
    last = len(Ms) - 1
    for j in range(len(Ms)):
        mj, ij = Ms[j], Is[j]
        swap = new_v < mj
        Ms[j] = jnp.where(swap, new_v, mj)
        Is[j] = jnp.where(swap, new_i, ij)
        if j != last:
            new_v = jnp.where(swap, mj, new_v)
            new_i = jnp.where(swap, ij, new_i)
    return Ms, Is


def _knn_ab_body(x_ref, xt_ref, w1_ref, b1_ref, idx_ref, a_ref, b_ref,
                 sq_scr, *, d_in, o1, o1g):
    i = pl.program_id(0)
    row0 = pl.multiple_of(i * R, R)
    x_r = x_ref[pl.ds(row0, R), :]
    x_r_bf = x_r.astype(jnp.bfloat16)
    xt_r = xt_ref[:, pl.ds(row0, R)]
    sq_r_row = jnp.sum(xt_r * xt_r, axis=0, keepdims=True)
    row_f = (lax.convert_element_type(i * R, jnp.float32)
             + lax.broadcasted_iota(jnp.int32, (1, R), 1
                                    ).astype(jnp.float32))
    sub8 = lax.broadcasted_iota(jnp.int32, (8, R), 0).astype(jnp.float32)

    @pl.when(i == 0)
    def _fill_sq():
        xf = x_ref[...]
        sq_scr[...] = jnp.sum(xf * xf, axis=1, keepdims=True)

    def make_d_t(c):
        off = pl.multiple_of(c * CBLK, CBLK)
        x_c_bf = x_ref[pl.ds(off, CBLK), :].astype(jnp.bfloat16)
        dots = lax.dot_general(x_c_bf, x_r_bf, (((1,), (1,)), ((), ())),
                               preferred_element_type=jnp.float32)
        sq_c = sq_scr[pl.ds(off, CBLK), :]
        return (sq_c + sq_r_row) - 2.0 * dots

    def sweep(depth):
        def chunk_body(c, carry):
            Ms, Is = carry
            Ms, Is = list(Ms), list(Is)
            d_t = make_d_t(c)
            basef = lax.convert_element_type(c * CBLK, jnp.float32)
            for t in range(CBLK // 8):
                cv = lax.slice(d_t, (8 * t, 0), (8 * t + 8, R))
                ci = sub8 + (basef + float(8 * t))
                cv = jnp.where(ci == row_f, jnp.inf, cv)
                Ms, Is = _insert_sorted(Ms, Is, cv, ci)
            return tuple(Ms), tuple(Is)

        Ms0 = tuple(jnp.full((8, R), jnp.inf, jnp.float32)
                    for _ in range(depth))
        Is0 = tuple(jnp.zeros((8, R), jnp.float32) for _ in range(depth))
        return lax.fori_loop(0, NPAD // CBLK, chunk_body, (Ms0, Is0))

    def merge(Ms, Is):
        V = jnp.concatenate(Ms, axis=0)
        Ic = jnp.concatenate(Is, axis=0)
        out_v, out_i = [], []
        for _ in range(K):
            m = jnp.min(V, axis=0, keepdims=True)
            am = jnp.min(jnp.where(V == m, Ic, jnp.float32(BIGF)),
                         axis=0, keepdims=True)
            am = jnp.minimum(am, jnp.float32(NPAD - 1))
            out_v.append(m)
            out_i.append(am)
            V = jnp.where(Ic == am, jnp.inf, V)
        return out_v, out_i

    def write_idx(out_i):
        outs = out_i + [out_i[-1], out_i[-1]]
        idx_ref[...] = jnp.concatenate(outs, axis=0).astype(jnp.int32)[None]

    Ms, Is = sweep(K)
    _, out_i = merge(Ms, Is)
    write_idx(out_i)

    wd = w1_ref[0:d_in, :] - w1_ref[d_in:2 * d_in, :]
    wb = w1_ref[d_in:2 * d_in, :]
    a_ref[...] = (jnp.dot(x_r, wd, preferred_element_type=jnp.float32,
                  precision=lax.Precision.HIGHEST)
                  + b1_ref[...])
    bm = jnp.dot(x_r, wb, preferred_element_type=jnp.float32,
                  precision=lax.Precision.HIGHEST)
    if o1g > o1:
        bm = jnp.concatenate(
            [bm, jnp.zeros((R, o1g - o1), jnp.float32)], axis=1)
    b_ref[...] = bm


def _knn_ab(x_p, x_t, w1, b1, d_in, o1, o1g, interpret=False):
    body = functools.partial(_knn_ab_body, d_in=d_in, o1=o1, o1g=o1g)
    return pl.pallas_call(
        body,
        grid=(NPAD // R,),
        in_specs=[
            pl.BlockSpec((NPAD, d_in), lambda i: (0, 0)),
            pl.BlockSpec((d_in, NPAD), lambda i: (0, 0)),
            pl.BlockSpec((2 * d_in, o1), lambda i: (0, 0)),
            pl.BlockSpec((1, o1), lambda i: (0, 0)),
        ],
        out_specs=[
            pl.BlockSpec((1, KPAD, R), lambda i: (i, 0, 0)),
            pl.BlockSpec((R, o1), lambda i: (i, 0)),
            pl.BlockSpec((R, o1g), lambda i: (i, 0)),
        ],
        out_shape=[
            jax.ShapeDtypeStruct((NPAD // R, KPAD, R), jnp.int32),
            jax.ShapeDtypeStruct((NPAD, o1), jnp.float32),
            jax.ShapeDtypeStruct((NPAD, o1g), jnp.float32),
        ],
        scratch_shapes=[pltpu.VMEM((NPAD, 1), jnp.float32)],
        interpret=interpret,
    )(x_p, x_t, w1, b1.reshape(1, o1))


def _sc_gather(b_mat, idx_flat, o1):
    nidx = idx_flat.shape[0]
    nw = SC_CORES * SC_SUBCORES
    per_w = nidx // nw
    n_chunks = per_w // SC_CHUNK
    mesh = plsc.VectorSubcoreMesh(core_axis_name="c", subcore_axis_name="s")

    @functools.partial(
        pl.kernel, mesh=mesh,
        out_type=jax.ShapeDtypeStruct((nidx, o1), jnp.float32),
        scratch_types=[
            pltpu.VMEM((per_w,), jnp.int32),
            pltpu.VMEM((SC_CHUNK, o1), jnp.float32),
            pltpu.VMEM((SC_CHUNK, o1), jnp.float32),
            pltpu.SemaphoreType.DMA,
            pltpu.SemaphoreType.DMA,
        ],
    )
    def gk(b_hbm, idx_hbm, out_hbm, idx_v, rows_v0, rows_v1, sem_g, sem_w):
        wid = lax.axis_index("s") * SC_CORES + lax.axis_index("c")
        base = wid * per_w
        pltpu.sync_copy(idx_hbm.at[pl.ds(base, per_w)], idx_v)
        bufs = (rows_v0, rows_v1)
        gathers = [None] * n_chunks
        writes = [None] * n_chunks
        gathers[0] = pltpu.async_copy(
            b_hbm.at[idx_v.at[pl.ds(0, SC_CHUNK)]], bufs[0], sem_g)
        for t in range(n_chunks):
            if t + 1 < n_chunks:
                if t >= 1:
                    writes[t - 1].wait()
                gathers[t + 1] = pltpu.async_copy(
                    b_hbm.at[idx_v.at[pl.ds((t + 1) * SC_CHUNK, SC_CHUNK)]],
                    bufs[(t + 1) % 2], sem_g)
            gathers[t].wait()
            writes[t] = pltpu.async_copy(
                bufs[t % 2], out_hbm.at[pl.ds(base + t * SC_CHUNK, SC_CHUNK)],
                sem_w)
        writes[n_chunks - 2].wait()
        writes[n_chunks - 1].wait()

    return gk(b_mat, idx_flat)


def _mlp_body(a_ref, g_ref, w2_ref, b2_ref, *rest, has_skip, o1, poison):
    if has_skip:
        s_ref, o_ref = rest
    else:
        (o_ref,) = rest
    a = a_ref[...]
    w2 = w2_ref[...]
    m = None
    for j in range(K):
        t = jnp.maximum(a + g_ref[j][:, 0:o1], 0.0)
        o = jnp.dot(t, w2, preferred_element_type=jnp.float32,
                  precision=lax.Precision.HIGHEST)
        m = o if m is None else jnp.maximum(m, o)
    m = m + b2_ref[...]
    if has_skip:
        m = m + s_ref[...]
    m = jnp.maximum(m, 0.0)
    if poison:
        rows = pl.program_id(0) * R + lax.broadcasted_iota(
            jnp.int32, (R, 1), 0)
        m = jnp.where(rows >= N, POISON, m)
    o_ref[...] = m


def _mlp(a, g, w2, b2, o1, o1g, o2, skip=None, poison=False,
         interpret=False):
    body = functools.partial(_mlp_body, has_skip=skip is not None, o1=o1,
                             poison=poison)
    in_specs = [
        pl.BlockSpec((R, o1), lambda i: (i, 0)),
        pl.BlockSpec((K, R, o1g), lambda i: (0, i, 0)),
        pl.BlockSpec((o1, o2), lambda i: (0, 0)),
        pl.BlockSpec((1, o2), lambda i: (0, 0)),
    ]
    args = [a, g, w2, b2.reshape(1, o2)]
    if skip is not None:
        in_specs.append(pl.BlockSpec((R, o2), lambda i: (i, 0)))
        args.append(skip)
    return pl.pallas_call(
        body,
        grid=(NPAD // R,),
        in_specs=in_specs,
        out_specs=pl.BlockSpec((R, o2), lambda i: (i, 0)),
        out_shape=jax.ShapeDtypeStruct((NPAD, o2), jnp.float32),
        interpret=interpret,
    )(*args)


def kernel(x, batch, W01, b01, W02, b02, W11, b11, W12, b12, W21, b21,
           W22, b22):
    x_p = jnp.pad(x, ((0, NPAD - N), (0, 0)), constant_values=POISON)

    def layer(xin, w1, b1, w2, b2, d_in, o1, o2, skip=None, poison=False):
        o1g = max(o1, 128)
        idx, a, bmat = _knn_ab(xin, xin.T, w1, b1, d_in, o1, o1g)
        idx_flat = idx[:, :K, :].transpose(1, 0, 2).reshape(-1)
        g = _sc_gather(bmat, idx_flat, o1g).reshape(K, NPAD, o1g)
        return _mlp(a, g, w2, b2, o1, o1g, o2, skip=skip, poison=poison)

    x0 = layer(x_p, W01, b01, W02, b02, 128, 128, 128, poison=True)
    x1 = layer(x0, W11, b11, W12, b12, 128, 64, 64, poison=True)
    out = layer(x1, W21, b21, W22, b22, 64, 128, 128, skip=x0)
    return out[:N]
